# baseline (speedup 1.0000x reference)
import functools

import jax
import jax.numpy as jnp
from jax import lax
from jax.experimental import pallas as pl
from jax.experimental.pallas import tpu as pltpu
from jax.experimental.pallas import tpu_sc as plsc

_NW = 32


def _iota16():
    return lax.iota(jnp.int32, 16)


def _table_transpose(tableT, v, d):
    nblk = v // 128
    mesh = plsc.VectorSubcoreMesh(core_axis_name="c", subcore_axis_name="s")

    @functools.partial(
        pl.kernel,
        mesh=mesh,
        compiler_params=pltpu.CompilerParams(
            use_tc_tiling_on_sc=True, needs_layout_passes=False
        ),
        out_type=jax.ShapeDtypeStruct((v * d,), jnp.float32),
        scratch_types=[
            pltpu.VMEM((8, 4, 8, 128), jnp.float32),
            pltpu.VMEM((32768,), jnp.float32),
            pltpu.SemaphoreType.DMA,
            pltpu.SemaphoreType.DMA,
        ],
    )
    def k1(t_hbm, out_hbm, ibuf, obuf, isem, osem):
        wid = lax.axis_index("s") * 2 + lax.axis_index("c")
        niter = (nblk + _NW - 1) // _NW

        iota = _iota16()
        dt_h = [jnp.full((16,), 2 * h, jnp.int32) + iota // 8 for h in (0, 1)]
        ds_v = iota % 8

        def in_copies(b, q):
            return [
                pltpu.make_async_copy(
                    t_hbm.at[pl.ds(8 * dt, 8), pl.ds(b * 128, 128)],
                    ibuf.at[q, dt],
                    isem,
                )
                for dt in range(4)
            ]

        def out_copy(b, q):
            return pltpu.make_async_copy(
                obuf.at[pl.ds(q * 4096, 4096)],
                out_hbm.at[pl.ds(b * 4096, 4096)],
                osem,
            )

        def valid(i):
            return (wid + i * _NW) < nblk

        def transpose(q):
            @plsc.parallel_loop(0, 8, unroll=4)
            def _(g):
                gc = jnp.full((16,), 16 * g, jnp.int32)
                for h in range(2):
                    st_base = gc + (512 * g - 16 * g + 16 * h) + iota
                    for r in range(16):
                        x = iota ^ r
                        cvec = gc + x
                        vals = plsc.load_gather(
                            ibuf.at[q], [dt_h[h], ds_v, cvec]
                        )
                        adst = st_base + (x << 5)
                        plsc.store_scatter(
                            obuf.at[pl.ds(q * 4096, 4096)], [adst], vals
                        )

        for u in range(8):
            @pl.when(valid(u))
            def _():
                for c in in_copies(wid + u * _NW, u):
                    c.start()

        @pl.loop(0, (niter + 7) // 8)
        def _(it):
            for u in range(8):
                i = 8 * it + u
                b = wid + i * _NW

                @pl.when(valid(i))
                def _():
                    for c in in_copies(b, u):
                        c.wait()

                    @pl.when(i >= 8)
                    def _():
                        out_copy(b, u).wait()

                    transpose(u)
                    out_copy(b, u).start()

                    @pl.when(valid(i + 8))
                    def _():
                        for c in in_copies(b + 8 * _NW, u):
                            c.start()

        for u in range(8):
            out_copy(0, u).wait()

    return k1(tableT)


def _gather_format(ids2, table_lin, b, s, d):
    mesh = plsc.VectorSubcoreMesh(core_axis_name="c", subcore_axis_name="s")

    @functools.partial(
        pl.kernel,
        mesh=mesh,
        compiler_params=pltpu.CompilerParams(
            use_tc_tiling_on_sc=False, needs_layout_passes=False
        ),
        out_type=jax.ShapeDtypeStruct((s, d // 8, b // 128, 1024), jnp.float32),
        scratch_types=[
            pltpu.VMEM((s, 128), jnp.int32),
            pltpu.VMEM((8, 128, d), jnp.float32),
            pltpu.VMEM((8, 4096), jnp.float32),
            pltpu.SemaphoreType.DMA,
            pltpu.SemaphoreType.DMA,
        ],
    )
    def k2(ids_hbm, t_hbm, out_hbm, idxs, rows, tbuf, gsem, ssem):
        wid = lax.axis_index("s") * 2 + lax.axis_index("c")

        iota = _iota16()
        dconst_h = [(iota + 16 * h) << 7 for h in (0, 1)]

        pltpu.sync_copy(ids_hbm.at[pl.ds(0, s), pl.ds(wid * 128, 128)], idxs)

        def gath(si, q):
            return pltpu.make_async_copy(t_hbm.at[idxs.at[si]], rows.at[q], gsem)

        def stores(si, q):
            return [
                pltpu.make_async_copy(
                    tbuf.at[q, pl.ds(1024 * dt, 1024)],
                    out_hbm.at[si, dt, wid],
                    ssem,
                )
                for dt in range(4)
            ]

        def transpose(q):
            @plsc.parallel_loop(0, 8, unroll=4)
            def _(g):
                gc = jnp.full((16,), 16 * g, jnp.int32)
                for h in range(2):
                    dvec = iota + 16 * h
                    st_base = dconst_h[h] + gc
                    for r in range(16):
                        x = iota ^ r
                        bvec = gc + x
                        vals = plsc.load_gather(rows.at[q], [bvec, dvec])
                        plsc.store_scatter(tbuf.at[q], [st_base + x], vals)

        for u in range(8):
            gath(u, u).start()

        @pl.loop(0, s // 8)
        def _(it):
            for u in range(8):
                si = 8 * it + u
                gath(si, u).wait()

                @pl.when(si >= 8)
                def _():
                    for c in stores(si, u):
                        c.wait()

                transpose(u)

                for c in stores(si, u):
                    c.start()

                @pl.when(si + 8 < s)
                def _():
                    gath(si + 8, u).start()

        for u in range(8):
            for c in stores(s - 8 + u, u):
                c.wait()

    return k2(ids2, table_lin)


def kernel(token_ids, table):
    b, s = token_ids.shape
    v, d = table.shape
    tableT = table.T
    ids2 = token_ids.T.astype(jnp.int32)
    lin = _table_transpose(tableT, v, d)
    table_lin = lin.reshape(v, d)
    out5 = _gather_format(ids2, table_lin, b, s, d)
    return (
        out5.reshape(s, d // 8, b // 128, 8, 128)
        .transpose(2, 4, 0, 1, 3)
        .reshape(b, s, d)
    )


# speedup vs baseline: 1.1521x; 1.1521x over previous
import functools

import jax
import jax.numpy as jnp
from jax import lax
from jax.experimental import pallas as pl
from jax.experimental.pallas import tpu as pltpu
from jax.experimental.pallas import tpu_sc as plsc

_NW = 32


def _iota16():
    return lax.iota(jnp.int32, 16)


def _table_transpose(tableT, v, d):
    nblk = v // 128
    mesh = plsc.VectorSubcoreMesh(core_axis_name="c", subcore_axis_name="s")

    @functools.partial(
        pl.kernel,
        mesh=mesh,
        compiler_params=pltpu.CompilerParams(
            use_tc_tiling_on_sc=True, needs_layout_passes=False
        ),
        out_type=jax.ShapeDtypeStruct((v * d,), jnp.float32),
        scratch_types=[
            pltpu.VMEM((4, 4, 8, 128), jnp.float32),
            pltpu.VMEM((16384,), jnp.float32),
            pltpu.SemaphoreType.DMA,
            pltpu.SemaphoreType.DMA,
        ],
    )
    def k1(t_hbm, out_hbm, ibuf, obuf, isem, osem):
        wid = lax.axis_index("s") * 2 + lax.axis_index("c")
        niter = (nblk + _NW - 1) // _NW

        iota = _iota16()
        dt_h = [jnp.full((16,), 2 * h, jnp.int32) + iota // 8 for h in (0, 1)]
        ds_v = iota % 8

        def in_copies(b, q):
            return [
                pltpu.make_async_copy(
                    t_hbm.at[pl.ds(8 * dt, 8), pl.ds(b * 128, 128)],
                    ibuf.at[q, dt],
                    isem,
                )
                for dt in range(4)
            ]

        def out_copy(b, q):
            return pltpu.make_async_copy(
                obuf.at[pl.ds(q * 4096, 4096)],
                out_hbm.at[pl.ds(b * 4096, 4096)],
                osem,
            )

        def valid(i):
            return (wid + i * _NW) < nblk

        def transpose(q):
            @plsc.parallel_loop(0, 8, unroll=4)
            def _(g):
                gc = jnp.full((16,), 16 * g, jnp.int32)
                for h in range(2):
                    st_base = gc + (512 * g - 16 * g + 16 * h) + iota
                    for r in range(16):
                        x = iota ^ r
                        cvec = gc + x
                        vals = plsc.load_gather(
                            ibuf.at[q], [dt_h[h], ds_v, cvec]
                        )
                        adst = st_base + (x << 5)
                        plsc.store_scatter(
                            obuf.at[pl.ds(q * 4096, 4096)], [adst], vals
                        )

        for u in range(4):
            @pl.when(valid(u))
            def _():
                for c in in_copies(wid + u * _NW, u):
                    c.start()

        @pl.loop(0, (niter + 3) // 4)
        def _(it):
            for u in range(4):
                i = 4 * it + u
                b = wid + i * _NW

                @pl.when(valid(i))
                def _():
                    for c in in_copies(b, u):
                        c.wait()

                    @pl.when(i >= 4)
                    def _():
                        out_copy(b, u).wait()

                    transpose(u)
                    out_copy(b, u).start()

                    @pl.when(valid(i + 4))
                    def _():
                        for c in in_copies(b + 4 * _NW, u):
                            c.start()

        for u in range(4):
            out_copy(0, u).wait()

    return k1(tableT)


def _gather_format(ids2, table_lin, b, s, d):
    mesh = plsc.VectorSubcoreMesh(core_axis_name="c", subcore_axis_name="s")

    @functools.partial(
        pl.kernel,
        mesh=mesh,
        compiler_params=pltpu.CompilerParams(
            use_tc_tiling_on_sc=False, needs_layout_passes=False
        ),
        out_type=jax.ShapeDtypeStruct((s, d // 8, b // 128, 1024), jnp.float32),
        scratch_types=[
            pltpu.VMEM((s, 128), jnp.int32),
            pltpu.VMEM((4, 128, d), jnp.float32),
            pltpu.VMEM((4, 4096), jnp.float32),
            pltpu.SemaphoreType.DMA,
            pltpu.SemaphoreType.DMA,
        ],
    )
    def k2(ids_hbm, t_hbm, out_hbm, idxs, rows, tbuf, gsem, ssem):
        wid = lax.axis_index("s") * 2 + lax.axis_index("c")

        iota = _iota16()
        dconst_h = [(iota + 16 * h) << 7 for h in (0, 1)]

        pltpu.sync_copy(ids_hbm.at[pl.ds(0, s), pl.ds(wid * 128, 128)], idxs)

        def gath(si, q):
            return pltpu.make_async_copy(t_hbm.at[idxs.at[si]], rows.at[q], gsem)

        def stores(si, q):
            return [
                pltpu.make_async_copy(
                    tbuf.at[q, pl.ds(1024 * dt, 1024)],
                    out_hbm.at[si, dt, wid],
                    ssem,
                )
                for dt in range(4)
            ]

        def transpose(q):
            @plsc.parallel_loop(0, 8, unroll=4)
            def _(g):
                gc = jnp.full((16,), 16 * g, jnp.int32)
                for h in range(2):
                    dvec = iota + 16 * h
                    st_base = dconst_h[h] + gc
                    for r in range(16):
                        x = iota ^ r
                        bvec = gc + x
                        vals = plsc.load_gather(rows.at[q], [bvec, dvec])
                        plsc.store_scatter(tbuf.at[q], [st_base + x], vals)

        for u in range(4):
            gath(u, u).start()

        @pl.loop(0, s // 4)
        def _(it):
            for u in range(4):
                si = 4 * it + u
                gath(si, u).wait()

                @pl.when(si >= 4)
                def _():
                    for c in stores(si, u):
                        c.wait()

                transpose(u)

                for c in stores(si, u):
                    c.start()

                @pl.when(si + 4 < s)
                def _():
                    gath(si + 4, u).start()

        for u in range(4):
            for c in stores(s - 4 + u, u):
                c.wait()

    return k2(ids2, table_lin)


def kernel(token_ids, table):
    b, s = token_ids.shape
    v, d = table.shape
    tableT = table.T
    ids2 = token_ids.T.astype(jnp.int32)
    lin = _table_transpose(tableT, v, d)
    table_lin = lin.reshape(v, d)
    out5 = _gather_format(ids2, table_lin, b, s, d)
    return (
        out5.reshape(s, d // 8, b // 128, 8, 128)
        .transpose(2, 4, 0, 1, 3)
        .reshape(b, s, d)
    )
